# Initial kernel scaffold; baseline (speedup 1.0000x reference)
import jax
import jax.numpy as jnp
from jax.experimental import pallas as pl


def kernel(x, norm1_g, norm1_b, score_w, score_b, wq, wk, wv, wo, bo, norm2_g, norm2_b, w1, b1, w2, b2):
    raise NotImplementedError("write your pallas kernel here")


# speedup vs baseline: 9.1182x; 9.1182x over previous
import functools

import jax
import jax.numpy as jnp
from jax import lax
from jax.experimental import pallas as pl
from jax.experimental.pallas import tpu as pltpu

_L = 144
_K = 5
_H = 6


def _stm_body(xn_ref, u_ref, sq_ref, sw_ref, sb_ref,
              wq_ref, wk_ref, wv_ref, wo_ref, bo_ref,
              n2g_ref, n2b_ref, w1_ref, b1_ref, w2_ref, b2_ref,
              out_ref, *, N, DIM, L, K, H):
    HD = DIM // H
    SCALE = HD ** -0.5
    f32 = jnp.float32

    ii = lax.broadcasted_iota(jnp.int32, (N, N), 0)
    jj = lax.broadcasted_iota(jnp.int32, (N, N), 1)
    diag = ii == jj

    def _to_row(v_col):
        e = jnp.where(diag, jnp.broadcast_to(v_col, (N, N)),
                      jnp.float32(-jnp.inf))
        return jnp.max(e, axis=0, keepdims=True)

    xn = xn_ref[0]
    sq_col = sq_ref[0]
    sq_row = _to_row(sq_col)

    ts_col = jnp.sum(xn * sw_ref[...], axis=-1, keepdims=True) + sb_ref[0, 0]

    g = lax.dot_general(xn, xn, (((1,), (1,)), ((), ())),
                        preferred_element_type=f32)
    d2 = sq_col + sq_row - 2.0 * g
    dm = jnp.sqrt(jnp.maximum(d2, 0.0)) / (DIM ** 0.5)

    cur = dm
    ms = []
    for _ in range(K):
        mn = jnp.min(cur, axis=-1, keepdims=True)
        ms.append(mn * mn)
        am = jnp.min(jnp.where(cur == mn, jj, N), axis=-1, keepdims=True)
        cur = jnp.where(jj == am, jnp.float32(jnp.inf), cur)
    acc = ((ms[0] + ms[4]) + ms[2]) + (ms[1] + ms[3])
    density_col = jnp.exp(-(acc / K)) + u_ref[0] * 1e-6
    density_row = _to_row(density_col)

    dist_max = jnp.max(dm)
    tmp = jnp.where(density_row > density_col, dm, dist_max)
    dist_col = jnp.min(tmp, axis=-1, keepdims=True)
    score_col = dist_col * density_col
    score_row = _to_row(score_col)

    gt = (score_row > score_col).astype(f32)
    tie = ((score_row == score_col) & (jj < ii)).astype(f32)
    rank_col = jnp.sum(gt + tie, axis=-1, keepdims=True)
    sel_col = rank_col < L
    rank_row = _to_row(rank_col)
    sel_row = rank_row < L

    cand = jnp.where(sel_row, dm, jnp.float32(jnp.inf))
    mind = jnp.min(cand, axis=-1, keepdims=True)
    cid_col = jnp.min(jnp.where(cand == mind, rank_row, jnp.float32(N)),
                      axis=-1, keepdims=True)
    cluster_col = jnp.where(sel_col, rank_col, cid_col)
    cluster_row = _to_row(cluster_col)

    w_col = jnp.exp(ts_col)
    w_row = _to_row(w_col)
    iota_c = lax.broadcasted_iota(jnp.int32, (L, N), 0).astype(f32)
    a = (cluster_row == iota_c).astype(f32)
    aw = a * w_row
    allw = jnp.sum(aw, axis=-1, keepdims=True) + 1e-6
    merged = lax.dot_general(aw, xn, (((1,), (0,)), ((), ())),
                             preferred_element_type=f32,
                             precision=lax.Precision.HIGHEST) / allw

    dn = (((1,), (0,)), ((), ()))
    q = lax.dot_general(merged, wq_ref[...], dn, preferred_element_type=f32)
    k = lax.dot_general(xn, wk_ref[...], dn, preferred_element_type=f32)
    v = lax.dot_general(xn, wv_ref[...], dn, preferred_element_type=f32)
    bias_row = _to_row(ts_col)
    outs = []
    for h in range(H):
        s = h * HD
        qh = q[:, s:s + HD]
        kh = k[:, s:s + HD]
        vh = v[:, s:s + HD]
        dots = lax.dot_general(qh, kh, (((1,), (1,)), ((), ())),
                               preferred_element_type=f32) * SCALE + bias_row
        mx = jnp.max(dots, axis=-1, keepdims=True)
        p = jnp.exp(dots - mx)
        attn = p / jnp.sum(p, axis=-1, keepdims=True)
        outs.append(lax.dot_general(attn, vh, dn, preferred_element_type=f32))
    att = jnp.concatenate(outs, axis=-1)
    att = lax.dot_general(att, wo_ref[...], dn,
                          preferred_element_type=f32) + bo_ref[...]

    feature = merged + att

    m2 = jnp.mean(feature, axis=-1, keepdims=True)
    v2 = jnp.mean((feature - m2) ** 2, axis=-1, keepdims=True)
    fn = (feature - m2) / jnp.sqrt(v2 + 1e-5) * n2g_ref[...] + n2b_ref[...]
    hh = lax.dot_general(fn, w1_ref[...], dn,
                         preferred_element_type=f32) + b1_ref[...]
    ge = 0.5 * hh * (1.0 + lax.erf(hh * (2.0 ** -0.5)))
    y = feature + lax.dot_general(ge, w2_ref[...], dn,
                                  preferred_element_type=f32) + b2_ref[...]

    out_ref[0] = y


def kernel(x, norm1_g, norm1_b, score_w, score_b, wq, wk, wv, wo, bo,
           norm2_g, norm2_b, w1, b1, w2, b2):
    B, N, DIM = x.shape
    L, K, H = _L, _K, _H
    HID = w1.shape[1]

    m = jnp.mean(x, axis=-1, keepdims=True)
    var = jnp.mean((x - m) ** 2, axis=-1, keepdims=True)
    xn = (x - m) / jnp.sqrt(var + 1e-5) * norm1_g + norm1_b
    sq = jnp.sum(xn * xn, axis=-1, keepdims=True)

    u = jax.random.uniform(jax.random.key(1), (B, N), jnp.float32)
    u3 = u.reshape(B, N, 1)

    sw_row = score_w.reshape(1, DIM)
    sb2 = score_b.reshape(1, 1)
    bo2 = bo.reshape(1, DIM)
    n2g2 = norm2_g.reshape(1, DIM)
    n2b2 = norm2_b.reshape(1, DIM)
    b1_2 = b1.reshape(1, HID)
    b2_2 = b2.reshape(1, DIM)

    def fixed(shape):
        nd = len(shape)
        return pl.BlockSpec(shape, lambda b, _nd=nd: (0,) * _nd)

    out = pl.pallas_call(
        functools.partial(_stm_body, N=N, DIM=DIM, L=L, K=K, H=H),
        grid=(B,),
        in_specs=[
            pl.BlockSpec((1, N, DIM), lambda b: (b, 0, 0)),
            pl.BlockSpec((1, N, 1), lambda b: (b, 0, 0)),
            pl.BlockSpec((1, N, 1), lambda b: (b, 0, 0)),
            fixed((1, DIM)),
            fixed((1, 1)),
            fixed((DIM, DIM)),
            fixed((DIM, DIM)),
            fixed((DIM, DIM)),
            fixed((DIM, DIM)),
            fixed((1, DIM)),
            fixed((1, DIM)),
            fixed((1, DIM)),
            fixed((DIM, HID)),
            fixed((1, HID)),
            fixed((HID, DIM)),
            fixed((1, DIM)),
        ],
        out_specs=pl.BlockSpec((1, L, DIM), lambda b: (b, 0, 0)),
        out_shape=jax.ShapeDtypeStruct((B, L, DIM), jnp.float32),
        compiler_params=pltpu.CompilerParams(
            dimension_semantics=("parallel",)),
    )(xn, u3, sq, sw_row, sb2, wq, wk, wv, wo, bo2,
      n2g2, n2b2, w1, b1_2, w2, b2_2)
    return out


# speedup vs baseline: 9.3953x; 1.0304x over previous
import functools

import jax
import jax.numpy as jnp
from jax import lax
from jax.experimental import pallas as pl
from jax.experimental.pallas import tpu as pltpu

_L = 144
_K = 5
_H = 6


def _stm_body(xn_ref, u_ref, sq_ref, sw_ref, sb_ref,
              wq_ref, wk_ref, wv_ref, wo_ref, bo_ref,
              n2g_ref, n2b_ref, w1_ref, b1_ref, w2_ref, b2_ref,
              out_ref, *, N, DIM, L, K, H):
    HD = DIM // H
    SCALE = HD ** -0.5
    f32 = jnp.float32

    ii = lax.broadcasted_iota(jnp.int32, (N, N), 0)
    jj = lax.broadcasted_iota(jnp.int32, (N, N), 1)
    diag = ii == jj

    def _to_row(v_col):
        e = jnp.where(diag, jnp.broadcast_to(v_col, (N, N)),
                      jnp.float32(-jnp.inf))
        return jnp.max(e, axis=0, keepdims=True)

    xn = xn_ref[0]
    sq_col = sq_ref[0]
    sq_row = _to_row(sq_col)

    ts_col = jnp.sum(xn * sw_ref[...], axis=-1, keepdims=True) + sb_ref[0, 0]

    g = lax.dot_general(xn, xn, (((1,), (1,)), ((), ())),
                        preferred_element_type=f32)
    d2 = sq_col + sq_row - 2.0 * g

    mns = [jnp.min(d2, axis=-1, keepdims=True)]
    d2max = jnp.max(d2)
    cnts = []
    for _ in range(K - 1):
        prev = mns[-1]
        cnts.append(jnp.sum((d2 == prev).astype(f32), axis=-1, keepdims=True))
        mns.append(jnp.min(jnp.where(d2 > prev, d2, jnp.float32(jnp.inf)),
                           axis=-1, keepdims=True))
    cum1 = cnts[0]
    cum2 = cum1 + cnts[1]
    cum3 = cum2 + cnts[2]
    cum4 = cum3 + cnts[3]

    def _pos(p):
        pf = jnp.float32(p)
        return jnp.where(pf < cum1, mns[0],
               jnp.where(pf < cum2, mns[1],
               jnp.where(pf < cum3, mns[2],
               jnp.where(pf < cum4, mns[3], mns[4]))))

    ms = []
    for p in range(K):
        s = jnp.sqrt(jnp.maximum(_pos(p), 0.0)) / (DIM ** 0.5)
        ms.append(s * s)
    acc = ((ms[0] + ms[4]) + ms[2]) + (ms[1] + ms[3])
    density_col = jnp.exp(-(acc / K)) + u_ref[0] * 1e-6
    density_row = _to_row(density_col)

    tmp = jnp.where(density_row > density_col, d2, d2max)
    dist_d2 = jnp.min(tmp, axis=-1, keepdims=True)
    dist_col = jnp.sqrt(jnp.maximum(dist_d2, 0.0)) / (DIM ** 0.5)
    score_col = dist_col * density_col
    score_row = _to_row(score_col)

    gt = (score_row > score_col).astype(f32)
    tie = ((score_row == score_col) & (jj < ii)).astype(f32)
    rank_col = jnp.sum(gt + tie, axis=-1, keepdims=True)
    sel_col = rank_col < L
    rank_row = _to_row(rank_col)
    sel_row = rank_row < L

    cand = jnp.where(sel_row,
                     jnp.sqrt(jnp.maximum(d2, 0.0)) / (DIM ** 0.5),
                     jnp.float32(jnp.inf))
    mind = jnp.min(cand, axis=-1, keepdims=True)
    cid_col = jnp.min(jnp.where(cand == mind, rank_row, jnp.float32(N)),
                      axis=-1, keepdims=True)
    cluster_col = jnp.where(sel_col, rank_col, cid_col)
    cluster_row = _to_row(cluster_col)

    bias_row = _to_row(ts_col)
    w_row = jnp.exp(bias_row)
    iota_c = lax.broadcasted_iota(jnp.int32, (L, N), 0).astype(f32)
    a = (cluster_row == iota_c).astype(f32)
    allw = jnp.sum(a * w_row, axis=-1, keepdims=True) + 1e-6
    ab = a.astype(jnp.bfloat16)
    xw = xn * jnp.exp(ts_col)
    xw_hi = xw.astype(jnp.bfloat16)
    xw_lo = (xw - xw_hi.astype(f32)).astype(jnp.bfloat16)
    dnm = (((1,), (0,)), ((), ()))
    merged = (lax.dot_general(ab, xw_hi, dnm, preferred_element_type=f32)
              + lax.dot_general(ab, xw_lo, dnm, preferred_element_type=f32)
              ) / allw

    dn = (((1,), (0,)), ((), ()))
    q = lax.dot_general(merged, wq_ref[...], dn, preferred_element_type=f32)
    k = lax.dot_general(xn, wk_ref[...], dn, preferred_element_type=f32)
    v = lax.dot_general(xn, wv_ref[...], dn, preferred_element_type=f32)
    outs = []
    for h in range(H):
        s = h * HD
        qh = q[:, s:s + HD]
        kh = k[:, s:s + HD]
        vh = v[:, s:s + HD]
        dots = lax.dot_general(qh, kh, (((1,), (1,)), ((), ())),
                               preferred_element_type=f32) * SCALE + bias_row
        mx = jnp.max(dots, axis=-1, keepdims=True)
        p = jnp.exp(dots - mx)
        attn = p / jnp.sum(p, axis=-1, keepdims=True)
        outs.append(lax.dot_general(attn, vh, dn, preferred_element_type=f32))
    att = jnp.concatenate(outs, axis=-1)
    att = lax.dot_general(att, wo_ref[...], dn,
                          preferred_element_type=f32) + bo_ref[...]

    feature = merged + att

    m2 = jnp.mean(feature, axis=-1, keepdims=True)
    v2 = jnp.mean((feature - m2) ** 2, axis=-1, keepdims=True)
    fn = (feature - m2) / jnp.sqrt(v2 + 1e-5) * n2g_ref[...] + n2b_ref[...]
    hh = lax.dot_general(fn, w1_ref[...], dn,
                         preferred_element_type=f32) + b1_ref[...]
    ge = 0.5 * hh * (1.0 + lax.erf(hh * (2.0 ** -0.5)))
    y = feature + lax.dot_general(ge, w2_ref[...], dn,
                                  preferred_element_type=f32) + b2_ref[...]

    out_ref[0] = y


def kernel(x, norm1_g, norm1_b, score_w, score_b, wq, wk, wv, wo, bo,
           norm2_g, norm2_b, w1, b1, w2, b2):
    B, N, DIM = x.shape
    L, K, H = _L, _K, _H
    HID = w1.shape[1]

    m = jnp.mean(x, axis=-1, keepdims=True)
    var = jnp.mean((x - m) ** 2, axis=-1, keepdims=True)
    xn = (x - m) / jnp.sqrt(var + 1e-5) * norm1_g + norm1_b
    sq = jnp.sum(xn * xn, axis=-1, keepdims=True)

    u = jax.random.uniform(jax.random.key(1), (B, N), jnp.float32)
    u3 = u.reshape(B, N, 1)

    sw_row = score_w.reshape(1, DIM)
    sb2 = score_b.reshape(1, 1)
    bo2 = bo.reshape(1, DIM)
    n2g2 = norm2_g.reshape(1, DIM)
    n2b2 = norm2_b.reshape(1, DIM)
    b1_2 = b1.reshape(1, HID)
    b2_2 = b2.reshape(1, DIM)

    def fixed(shape):
        nd = len(shape)
        return pl.BlockSpec(shape, lambda b, _nd=nd: (0,) * _nd)

    out = pl.pallas_call(
        functools.partial(_stm_body, N=N, DIM=DIM, L=L, K=K, H=H),
        grid=(B,),
        in_specs=[
            pl.BlockSpec((1, N, DIM), lambda b: (b, 0, 0)),
            pl.BlockSpec((1, N, 1), lambda b: (b, 0, 0)),
            pl.BlockSpec((1, N, 1), lambda b: (b, 0, 0)),
            fixed((1, DIM)),
            fixed((1, 1)),
            fixed((DIM, DIM)),
            fixed((DIM, DIM)),
            fixed((DIM, DIM)),
            fixed((DIM, DIM)),
            fixed((1, DIM)),
            fixed((1, DIM)),
            fixed((1, DIM)),
            fixed((DIM, HID)),
            fixed((1, HID)),
            fixed((HID, DIM)),
            fixed((1, DIM)),
        ],
        out_specs=pl.BlockSpec((1, L, DIM), lambda b: (b, 0, 0)),
        out_shape=jax.ShapeDtypeStruct((B, L, DIM), jnp.float32),
        compiler_params=pltpu.CompilerParams(
            dimension_semantics=("parallel",)),
    )(xn, u3, sq, sw_row, sb2, wq, wk, wv, wo, bo2,
      n2g2, n2b2, w1, b1_2, w2, b2_2)
    return out


# speedup vs baseline: 9.4008x; 1.0006x over previous
import functools

import jax
import jax.numpy as jnp
from jax import lax
from jax.experimental import pallas as pl
from jax.experimental.pallas import tpu as pltpu

_L = 144
_K = 5
_H = 6


def _stm_body(xn_ref, u_ref, sq_ref, sw_ref, sb_ref,
              wq_ref, wk_ref, wv_ref, wo_ref, bo_ref,
              n2g_ref, n2b_ref, w1_ref, b1_ref, w2_ref, b2_ref,
              out_ref, *, N, DIM, L, K, H, SPB):
    HD = DIM // H
    SCALE = HD ** -0.5
    f32 = jnp.float32

    ii = lax.broadcasted_iota(jnp.int32, (N, N), 0)
    jj = lax.broadcasted_iota(jnp.int32, (N, N), 1)
    diag = ii == jj

    def _to_row(v_col):
        e = jnp.where(diag, jnp.broadcast_to(v_col, (N, N)),
                      jnp.float32(-jnp.inf))
        return jnp.max(e, axis=0, keepdims=True)

    for smp in range(SPB):
        _one_sample(smp, xn_ref, u_ref, sq_ref, sw_ref, sb_ref,
                    wq_ref, wk_ref, wv_ref, wo_ref, bo_ref,
                    n2g_ref, n2b_ref, w1_ref, b1_ref, w2_ref, b2_ref,
                    out_ref, N, DIM, L, K, H, HD, SCALE, f32,
                    ii, jj, _to_row)


def _one_sample(smp, xn_ref, u_ref, sq_ref, sw_ref, sb_ref,
                wq_ref, wk_ref, wv_ref, wo_ref, bo_ref,
                n2g_ref, n2b_ref, w1_ref, b1_ref, w2_ref, b2_ref,
                out_ref, N, DIM, L, K, H, HD, SCALE, f32,
                ii, jj, _to_row):
    xn = xn_ref[smp]
    sq_col = sq_ref[smp]
    sq_row = _to_row(sq_col)

    ts_col = jnp.sum(xn * sw_ref[...], axis=-1, keepdims=True) + sb_ref[0, 0]

    g = lax.dot_general(xn, xn, (((1,), (1,)), ((), ())),
                        preferred_element_type=f32)
    d2 = sq_col + sq_row - 2.0 * g

    mns = [jnp.min(d2, axis=-1, keepdims=True)]
    d2max = jnp.max(d2)
    for _ in range(K - 1):
        prev = mns[-1]
        mns.append(jnp.min(jnp.where(d2 > prev, d2, jnp.float32(jnp.inf)),
                           axis=-1, keepdims=True))
    cnts = [jnp.sum((d2 == mns[i]).astype(f32), axis=-1, keepdims=True)
            for i in range(K - 1)]
    cum1 = cnts[0]
    cum2 = cum1 + cnts[1]
    cum3 = cum2 + cnts[2]
    cum4 = cum3 + cnts[3]

    def _pos(p):
        pf = jnp.float32(p)
        return jnp.where(pf < cum1, mns[0],
               jnp.where(pf < cum2, mns[1],
               jnp.where(pf < cum3, mns[2],
               jnp.where(pf < cum4, mns[3], mns[4]))))

    ms = []
    for p in range(K):
        s = jnp.sqrt(jnp.maximum(_pos(p), 0.0)) / (DIM ** 0.5)
        ms.append(s * s)
    acc = ((ms[0] + ms[4]) + ms[2]) + (ms[1] + ms[3])
    density_col = jnp.exp(-(acc / K)) + u_ref[smp] * 1e-6
    density_row = _to_row(density_col)

    tmp = jnp.where(density_row > density_col, d2, d2max)
    dist_d2 = jnp.min(tmp, axis=-1, keepdims=True)
    dist_col = jnp.sqrt(jnp.maximum(dist_d2, 0.0)) / (DIM ** 0.5)
    score_col = dist_col * density_col
    score_row = _to_row(score_col)

    gt = (score_row > score_col).astype(f32)
    tie = ((score_row == score_col) & (jj < ii)).astype(f32)
    rank_col = jnp.sum(gt + tie, axis=-1, keepdims=True)
    sel_col = rank_col < L
    rank_row = _to_row(rank_col)
    sel_row = rank_row < L

    cand = jnp.where(sel_row,
                     jnp.sqrt(jnp.maximum(d2, 0.0)) / (DIM ** 0.5),
                     jnp.float32(jnp.inf))
    mind = jnp.min(cand, axis=-1, keepdims=True)
    cid_col = jnp.min(jnp.where(cand == mind, rank_row, jnp.float32(N)),
                      axis=-1, keepdims=True)
    cluster_col = jnp.where(sel_col, rank_col, cid_col)
    cluster_row = _to_row(cluster_col)

    bias_row = _to_row(ts_col)
    w_row = jnp.exp(bias_row)
    iota_c = lax.broadcasted_iota(jnp.int32, (L, N), 0).astype(f32)
    a = (cluster_row == iota_c).astype(f32)
    allw = jnp.sum(a * w_row, axis=-1, keepdims=True) + 1e-6
    ab = a.astype(jnp.bfloat16)
    xw = xn * jnp.exp(ts_col)
    xw_hi = xw.astype(jnp.bfloat16)
    xw_lo = (xw - xw_hi.astype(f32)).astype(jnp.bfloat16)
    dnm = (((1,), (0,)), ((), ()))
    merged = (lax.dot_general(ab, xw_hi, dnm, preferred_element_type=f32)
              + lax.dot_general(ab, xw_lo, dnm, preferred_element_type=f32)
              ) / allw

    dn = (((1,), (0,)), ((), ()))
    q = lax.dot_general(merged, wq_ref[...], dn, preferred_element_type=f32)
    k = lax.dot_general(xn, wk_ref[...], dn, preferred_element_type=f32)
    v = lax.dot_general(xn, wv_ref[...], dn, preferred_element_type=f32)
    outs = []
    for h in range(H):
        s = h * HD
        qh = q[:, s:s + HD]
        kh = k[:, s:s + HD]
        vh = v[:, s:s + HD]
        dots = lax.dot_general(qh, kh, (((1,), (1,)), ((), ())),
                               preferred_element_type=f32) * SCALE + bias_row
        mx = jnp.max(dots, axis=-1, keepdims=True)
        p = jnp.exp(dots - mx)
        attn = p / jnp.sum(p, axis=-1, keepdims=True)
        outs.append(lax.dot_general(attn, vh, dn, preferred_element_type=f32))
    att = jnp.concatenate(outs, axis=-1)
    att = lax.dot_general(att, wo_ref[...], dn,
                          preferred_element_type=f32) + bo_ref[...]

    feature = merged + att

    m2 = jnp.mean(feature, axis=-1, keepdims=True)
    v2 = jnp.mean((feature - m2) ** 2, axis=-1, keepdims=True)
    fn = (feature - m2) / jnp.sqrt(v2 + 1e-5) * n2g_ref[...] + n2b_ref[...]
    hh = lax.dot_general(fn, w1_ref[...], dn,
                         preferred_element_type=f32) + b1_ref[...]
    ge = 0.5 * hh * (1.0 + lax.erf(hh * (2.0 ** -0.5)))
    y = feature + lax.dot_general(ge, w2_ref[...], dn,
                                  preferred_element_type=f32) + b2_ref[...]

    out_ref[smp] = y


def kernel(x, norm1_g, norm1_b, score_w, score_b, wq, wk, wv, wo, bo,
           norm2_g, norm2_b, w1, b1, w2, b2):
    B, N, DIM = x.shape
    L, K, H = _L, _K, _H
    HID = w1.shape[1]

    m = jnp.mean(x, axis=-1, keepdims=True)
    var = jnp.mean((x - m) ** 2, axis=-1, keepdims=True)
    xn = (x - m) / jnp.sqrt(var + 1e-5) * norm1_g + norm1_b
    sq = jnp.sum(xn * xn, axis=-1, keepdims=True)

    u = jax.random.uniform(jax.random.key(1), (B, N), jnp.float32)
    u3 = u.reshape(B, N, 1)

    sw_row = score_w.reshape(1, DIM)
    sb2 = score_b.reshape(1, 1)
    bo2 = bo.reshape(1, DIM)
    n2g2 = norm2_g.reshape(1, DIM)
    n2b2 = norm2_b.reshape(1, DIM)
    b1_2 = b1.reshape(1, HID)
    b2_2 = b2.reshape(1, DIM)

    def fixed(shape):
        nd = len(shape)
        return pl.BlockSpec(shape, lambda b, _nd=nd: (0,) * _nd)

    SPB = 2
    out = pl.pallas_call(
        functools.partial(_stm_body, N=N, DIM=DIM, L=L, K=K, H=H, SPB=SPB),
        grid=(B // SPB,),
        in_specs=[
            pl.BlockSpec((SPB, N, DIM), lambda b: (b, 0, 0)),
            pl.BlockSpec((SPB, N, 1), lambda b: (b, 0, 0)),
            pl.BlockSpec((SPB, N, 1), lambda b: (b, 0, 0)),
            fixed((1, DIM)),
            fixed((1, 1)),
            fixed((DIM, DIM)),
            fixed((DIM, DIM)),
            fixed((DIM, DIM)),
            fixed((DIM, DIM)),
            fixed((1, DIM)),
            fixed((1, DIM)),
            fixed((1, DIM)),
            fixed((DIM, HID)),
            fixed((1, HID)),
            fixed((HID, DIM)),
            fixed((1, DIM)),
        ],
        out_specs=pl.BlockSpec((SPB, L, DIM), lambda b: (b, 0, 0)),
        out_shape=jax.ShapeDtypeStruct((B, L, DIM), jnp.float32),
        compiler_params=pltpu.CompilerParams(
            dimension_semantics=("parallel",)),
    )(xn, u3, sq, sw_row, sb2, wq, wk, wv, wo, bo2,
      n2g2, n2b2, w1, b1_2, w2, b2_2)
    return out


# speedup vs baseline: 10.7665x; 1.1453x over previous
import functools

import jax
import jax.numpy as jnp
from jax import lax
from jax.experimental import pallas as pl
from jax.experimental.pallas import tpu as pltpu

_L = 144
_K = 5
_H = 6


def _stm_body(xn_ref, u_ref, sq_ref, sw_ref, sb_ref,
              wq_ref, wk_ref, wv_ref, wo_ref, bo_ref,
              n2g_ref, n2b_ref, w1_ref, b1_ref, w2_ref, b2_ref,
              out_ref, *, N, DIM, L, K, H, SPB):
    HD = DIM // H
    SCALE = HD ** -0.5
    f32 = jnp.float32

    ii = lax.broadcasted_iota(jnp.int32, (N, N), 0)
    jj = lax.broadcasted_iota(jnp.int32, (N, N), 1)
    diag = ii == jj

    def _to_row(v_col):
        e = jnp.where(diag, jnp.broadcast_to(v_col, (N, N)),
                      jnp.float32(-jnp.inf))
        return jnp.max(e, axis=0, keepdims=True)

    for smp in range(SPB):
        _one_sample(smp, xn_ref, u_ref, sq_ref, sw_ref, sb_ref,
                    wq_ref, wk_ref, wv_ref, wo_ref, bo_ref,
                    n2g_ref, n2b_ref, w1_ref, b1_ref, w2_ref, b2_ref,
                    out_ref, N, DIM, L, K, H, HD, SCALE, f32,
                    ii, jj, _to_row)


def _one_sample(smp, xn_ref, u_ref, sq_ref, sw_ref, sb_ref,
                wq_ref, wk_ref, wv_ref, wo_ref, bo_ref,
                n2g_ref, n2b_ref, w1_ref, b1_ref, w2_ref, b2_ref,
                out_ref, N, DIM, L, K, H, HD, SCALE, f32,
                ii, jj, _to_row):
    diag = ii == jj
    xn = xn_ref[smp]
    sq_col = sq_ref[smp]
    sq_row = _to_row(sq_col)
    u_row = u_ref[smp]

    def _to_col(v_row):
        e = jnp.where(diag, jnp.broadcast_to(v_row, (N, N)),
                      jnp.float32(jnp.inf))
        return jnp.min(e, axis=-1, keepdims=True)

    ts_col = jnp.sum(xn * sw_ref[...], axis=-1, keepdims=True) + sb_ref[0, 0]

    g = lax.dot_general(xn, xn, (((1,), (1,)), ((), ())),
                        preferred_element_type=f32)
    d2 = sq_col + sq_row - 2.0 * g

    mns = [jnp.min(d2, axis=0, keepdims=True)]
    d2max = jnp.max(d2)
    for _ in range(K - 1):
        prev = mns[-1]
        mns.append(jnp.min(jnp.where(d2 > prev, d2, jnp.float32(jnp.inf)),
                           axis=0, keepdims=True))
    cnts = [jnp.sum((d2 == mns[i]).astype(f32), axis=0, keepdims=True)
            for i in range(K - 1)]
    cum1 = cnts[0]
    cum2 = cum1 + cnts[1]
    cum3 = cum2 + cnts[2]
    cum4 = cum3 + cnts[3]

    def _pos(p):
        pf = jnp.float32(p)
        return jnp.where(pf < cum1, mns[0],
               jnp.where(pf < cum2, mns[1],
               jnp.where(pf < cum3, mns[2],
               jnp.where(pf < cum4, mns[3], mns[4]))))

    ms = []
    for p in range(K):
        s = jnp.sqrt(jnp.maximum(_pos(p), 0.0)) / (DIM ** 0.5)
        ms.append(s * s)
    acc = ((ms[0] + ms[4]) + ms[2]) + (ms[1] + ms[3])
    density_row = jnp.exp(-(acc / K)) + u_row * 1e-6
    density_col = _to_col(density_row)

    tmp = jnp.where(density_col > density_row, d2, d2max)
    dist_d2 = jnp.min(tmp, axis=0, keepdims=True)
    dist_row = jnp.sqrt(jnp.maximum(dist_d2, 0.0)) / (DIM ** 0.5)
    score_row = dist_row * density_row
    score_col = _to_col(score_row)

    gt = (score_col > score_row).astype(f32)
    tie = ((score_col == score_row) & (ii < jj)).astype(f32)
    rank_row = jnp.sum(gt + tie, axis=0, keepdims=True)
    sel_row = rank_row < L
    rank_col = _to_col(rank_row)
    sel_col = rank_col < L

    cand = jnp.where(sel_col,
                     jnp.sqrt(jnp.maximum(d2, 0.0)) / (DIM ** 0.5),
                     jnp.float32(jnp.inf))
    mind = jnp.min(cand, axis=0, keepdims=True)
    cid_row = jnp.min(jnp.where(cand == mind, rank_col, jnp.float32(N)),
                      axis=0, keepdims=True)
    cluster_row = jnp.where(sel_row, rank_row, cid_row)

    bias_row = _to_row(ts_col)
    w_row = jnp.exp(bias_row)
    iota_c = lax.broadcasted_iota(jnp.int32, (L, N), 0).astype(f32)
    a = (cluster_row == iota_c).astype(f32)
    allw = jnp.sum(a * w_row, axis=-1, keepdims=True) + 1e-6
    ab = a.astype(jnp.bfloat16)
    xw = xn * jnp.exp(ts_col)
    xw_hi = xw.astype(jnp.bfloat16)
    xw_lo = (xw - xw_hi.astype(f32)).astype(jnp.bfloat16)
    dnm = (((1,), (0,)), ((), ()))
    merged = (lax.dot_general(ab, xw_hi, dnm, preferred_element_type=f32)
              + lax.dot_general(ab, xw_lo, dnm, preferred_element_type=f32)
              ) / allw

    dn = (((1,), (0,)), ((), ()))
    q = lax.dot_general(merged, wq_ref[...], dn, preferred_element_type=f32)
    k = lax.dot_general(xn, wk_ref[...], dn, preferred_element_type=f32)
    v = lax.dot_general(xn, wv_ref[...], dn, preferred_element_type=f32)
    outs = []
    for h in range(H):
        s = h * HD
        qh = q[:, s:s + HD]
        kh = k[:, s:s + HD]
        vh = v[:, s:s + HD]
        dots = lax.dot_general(qh, kh, (((1,), (1,)), ((), ())),
                               preferred_element_type=f32) * SCALE + bias_row
        mx = jnp.max(dots, axis=-1, keepdims=True)
        p = jnp.exp(dots - mx)
        attn = p / jnp.sum(p, axis=-1, keepdims=True)
        outs.append(lax.dot_general(attn, vh, dn, preferred_element_type=f32))
    att = jnp.concatenate(outs, axis=-1)
    att = lax.dot_general(att, wo_ref[...], dn,
                          preferred_element_type=f32) + bo_ref[...]

    feature = merged + att

    m2 = jnp.mean(feature, axis=-1, keepdims=True)
    v2 = jnp.mean((feature - m2) ** 2, axis=-1, keepdims=True)
    fn = (feature - m2) / jnp.sqrt(v2 + 1e-5) * n2g_ref[...] + n2b_ref[...]
    hh = lax.dot_general(fn, w1_ref[...], dn,
                         preferred_element_type=f32) + b1_ref[...]
    ge = 0.5 * hh * (1.0 + lax.erf(hh * (2.0 ** -0.5)))
    y = feature + lax.dot_general(ge, w2_ref[...], dn,
                                  preferred_element_type=f32) + b2_ref[...]

    out_ref[smp] = y


def kernel(x, norm1_g, norm1_b, score_w, score_b, wq, wk, wv, wo, bo,
           norm2_g, norm2_b, w1, b1, w2, b2):
    B, N, DIM = x.shape
    L, K, H = _L, _K, _H
    HID = w1.shape[1]

    m = jnp.mean(x, axis=-1, keepdims=True)
    var = jnp.mean((x - m) ** 2, axis=-1, keepdims=True)
    xn = (x - m) / jnp.sqrt(var + 1e-5) * norm1_g + norm1_b
    sq = jnp.sum(xn * xn, axis=-1, keepdims=True)

    u = jax.random.uniform(jax.random.key(1), (B, N), jnp.float32)
    u3 = u.reshape(B, 1, N)

    sw_row = score_w.reshape(1, DIM)
    sb2 = score_b.reshape(1, 1)
    bo2 = bo.reshape(1, DIM)
    n2g2 = norm2_g.reshape(1, DIM)
    n2b2 = norm2_b.reshape(1, DIM)
    b1_2 = b1.reshape(1, HID)
    b2_2 = b2.reshape(1, DIM)

    def fixed(shape):
        nd = len(shape)
        return pl.BlockSpec(shape, lambda b, _nd=nd: (0,) * _nd)

    SPB = 2
    out = pl.pallas_call(
        functools.partial(_stm_body, N=N, DIM=DIM, L=L, K=K, H=H, SPB=SPB),
        grid=(B // SPB,),
        in_specs=[
            pl.BlockSpec((SPB, N, DIM), lambda b: (b, 0, 0)),
            pl.BlockSpec((SPB, 1, N), lambda b: (b, 0, 0)),
            pl.BlockSpec((SPB, N, 1), lambda b: (b, 0, 0)),
            fixed((1, DIM)),
            fixed((1, 1)),
            fixed((DIM, DIM)),
            fixed((DIM, DIM)),
            fixed((DIM, DIM)),
            fixed((DIM, DIM)),
            fixed((1, DIM)),
            fixed((1, DIM)),
            fixed((1, DIM)),
            fixed((DIM, HID)),
            fixed((1, HID)),
            fixed((HID, DIM)),
            fixed((1, DIM)),
        ],
        out_specs=pl.BlockSpec((SPB, L, DIM), lambda b: (b, 0, 0)),
        out_shape=jax.ShapeDtypeStruct((B, L, DIM), jnp.float32),
        compiler_params=pltpu.CompilerParams(
            dimension_semantics=("parallel",)),
    )(xn, u3, sq, sw_row, sb2, wq, wk, wv, wo, bo2,
      n2g2, n2b2, w1, b1_2, w2, b2_2)
    return out


# speedup vs baseline: 10.9865x; 1.0204x over previous
import functools

import jax
import jax.numpy as jnp
from jax import lax
from jax.experimental import pallas as pl
from jax.experimental.pallas import tpu as pltpu

_L = 144
_K = 5
_H = 6


def _stm_body(xn_ref, u_ref, sq_ref, sw_ref, sb_ref,
              wq_ref, wk_ref, wv_ref, wo_ref, bo_ref,
              n2g_ref, n2b_ref, w1_ref, b1_ref, w2_ref, b2_ref,
              out_ref, *, N, DIM, L, K, H, SPB):
    HD = DIM // H
    SCALE = HD ** -0.5
    f32 = jnp.float32

    ii = lax.broadcasted_iota(jnp.int32, (N, N), 0)
    jj = lax.broadcasted_iota(jnp.int32, (N, N), 1)
    diag = ii == jj

    def _to_row(v_col):
        e = jnp.where(diag, jnp.broadcast_to(v_col, (N, N)),
                      jnp.float32(-jnp.inf))
        return jnp.max(e, axis=0, keepdims=True)

    for smp in range(SPB):
        _one_sample(smp, xn_ref, u_ref, sq_ref, sw_ref, sb_ref,
                    wq_ref, wk_ref, wv_ref, wo_ref, bo_ref,
                    n2g_ref, n2b_ref, w1_ref, b1_ref, w2_ref, b2_ref,
                    out_ref, N, DIM, L, K, H, HD, SCALE, f32,
                    ii, jj, _to_row)


def _one_sample(smp, xn_ref, u_ref, sq_ref, sw_ref, sb_ref,
                wq_ref, wk_ref, wv_ref, wo_ref, bo_ref,
                n2g_ref, n2b_ref, w1_ref, b1_ref, w2_ref, b2_ref,
                out_ref, N, DIM, L, K, H, HD, SCALE, f32,
                ii, jj, _to_row):
    diag = ii == jj
    xn = xn_ref[smp]
    sq_col = sq_ref[smp]
    sq_row = _to_row(sq_col)
    u_row = u_ref[smp]

    def _to_col(v_row):
        e = jnp.where(diag, jnp.broadcast_to(v_row, (N, N)),
                      jnp.float32(jnp.inf))
        return jnp.min(e, axis=-1, keepdims=True)

    ts_col = jnp.sum(xn * sw_ref[...], axis=-1, keepdims=True) + sb_ref[0, 0]

    g = lax.dot_general(xn, xn, (((1,), (1,)), ((), ())),
                        preferred_element_type=f32)
    d2 = sq_col + sq_row - 2.0 * g

    mns = [jnp.min(d2, axis=0, keepdims=True)]
    d2max = jnp.max(d2)
    for _ in range(K - 1):
        prev = mns[-1]
        mns.append(jnp.min(jnp.where(d2 > prev, d2, jnp.float32(jnp.inf)),
                           axis=0, keepdims=True))
    enc12 = jnp.sum((d2 == mns[0]).astype(f32)
                    + 1024.0 * (d2 == mns[1]).astype(f32),
                    axis=0, keepdims=True)
    enc34 = jnp.sum((d2 == mns[2]).astype(f32)
                    + 1024.0 * (d2 == mns[3]).astype(f32),
                    axis=0, keepdims=True)
    c2 = jnp.floor(enc12 * (1.0 / 1024.0))
    c1 = enc12 - 1024.0 * c2
    c4 = jnp.floor(enc34 * (1.0 / 1024.0))
    c3 = enc34 - 1024.0 * c4
    cum1 = c1
    cum2 = cum1 + c2
    cum3 = cum2 + c3
    cum4 = cum3 + c4

    def _pos(p):
        pf = jnp.float32(p)
        return jnp.where(pf < cum1, mns[0],
               jnp.where(pf < cum2, mns[1],
               jnp.where(pf < cum3, mns[2],
               jnp.where(pf < cum4, mns[3], mns[4]))))

    ms = []
    for p in range(K):
        s = jnp.sqrt(jnp.maximum(_pos(p), 0.0)) / (DIM ** 0.5)
        ms.append(s * s)
    acc = ((ms[0] + ms[4]) + ms[2]) + (ms[1] + ms[3])
    density_row = jnp.exp(-(acc / K)) + u_row * 1e-6
    density_col = _to_col(density_row)

    tmp = jnp.where(density_col > density_row, d2, d2max)
    dist_d2 = jnp.min(tmp, axis=0, keepdims=True)
    dist_row = jnp.sqrt(jnp.maximum(dist_d2, 0.0)) / (DIM ** 0.5)
    score_row = dist_row * density_row
    score_col = _to_col(score_row)

    gt = (score_col > score_row).astype(f32)
    tie = ((score_col == score_row) & (ii < jj)).astype(f32)
    rank_row = jnp.sum(gt + tie, axis=0, keepdims=True)
    sel_row = rank_row < L
    rank_col = _to_col(rank_row)
    sel_col = rank_col < L

    cand = jnp.where(sel_col,
                     jnp.sqrt(jnp.maximum(d2, 0.0)) / (DIM ** 0.5),
                     jnp.float32(jnp.inf))
    mind = jnp.min(cand, axis=0, keepdims=True)
    cid_row = jnp.min(jnp.where(cand == mind, rank_col, jnp.float32(N)),
                      axis=0, keepdims=True)
    cluster_row = jnp.where(sel_row, rank_row, cid_row)

    bias_row = _to_row(ts_col)
    w_row = jnp.exp(bias_row)
    iota_c = lax.broadcasted_iota(jnp.int32, (L, N), 0).astype(f32)
    a = (cluster_row == iota_c).astype(f32)
    allw = jnp.sum(a * w_row, axis=-1, keepdims=True) + 1e-6
    xw = xn * jnp.exp(ts_col)
    dnm = (((1,), (0,)), ((), ()))
    merged = lax.dot_general(a, xw, dnm, preferred_element_type=f32) / allw

    dn = (((1,), (0,)), ((), ()))
    q = lax.dot_general(merged, wq_ref[...], dn, preferred_element_type=f32)
    k = lax.dot_general(xn, wk_ref[...], dn, preferred_element_type=f32)
    v = lax.dot_general(xn, wv_ref[...], dn, preferred_element_type=f32)
    outs = []
    for h in range(H):
        s = h * HD
        qh = q[:, s:s + HD]
        kh = k[:, s:s + HD]
        vh = v[:, s:s + HD]
        dots = lax.dot_general(qh, kh, (((1,), (1,)), ((), ())),
                               preferred_element_type=f32) * SCALE + bias_row
        mx = jnp.max(dots, axis=-1, keepdims=True)
        p = jnp.exp(dots - mx)
        attn = p / jnp.sum(p, axis=-1, keepdims=True)
        outs.append(lax.dot_general(attn, vh, dn, preferred_element_type=f32))
    att = jnp.concatenate(outs, axis=-1)
    att = lax.dot_general(att, wo_ref[...], dn,
                          preferred_element_type=f32) + bo_ref[...]

    feature = merged + att

    m2 = jnp.mean(feature, axis=-1, keepdims=True)
    v2 = jnp.mean((feature - m2) ** 2, axis=-1, keepdims=True)
    fn = (feature - m2) / jnp.sqrt(v2 + 1e-5) * n2g_ref[...] + n2b_ref[...]
    hh = lax.dot_general(fn, w1_ref[...], dn,
                         preferred_element_type=f32) + b1_ref[...]
    ge = 0.5 * hh * (1.0 + lax.erf(hh * (2.0 ** -0.5)))
    y = feature + lax.dot_general(ge, w2_ref[...], dn,
                                  preferred_element_type=f32) + b2_ref[...]

    out_ref[smp] = y


def kernel(x, norm1_g, norm1_b, score_w, score_b, wq, wk, wv, wo, bo,
           norm2_g, norm2_b, w1, b1, w2, b2):
    B, N, DIM = x.shape
    L, K, H = _L, _K, _H
    HID = w1.shape[1]

    m = jnp.mean(x, axis=-1, keepdims=True)
    var = jnp.mean((x - m) ** 2, axis=-1, keepdims=True)
    xn = (x - m) / jnp.sqrt(var + 1e-5) * norm1_g + norm1_b
    sq = jnp.sum(xn * xn, axis=-1, keepdims=True)

    u = jax.random.uniform(jax.random.key(1), (B, N), jnp.float32)
    u3 = u.reshape(B, 1, N)

    sw_row = score_w.reshape(1, DIM)
    sb2 = score_b.reshape(1, 1)
    bo2 = bo.reshape(1, DIM)
    n2g2 = norm2_g.reshape(1, DIM)
    n2b2 = norm2_b.reshape(1, DIM)
    b1_2 = b1.reshape(1, HID)
    b2_2 = b2.reshape(1, DIM)

    def fixed(shape):
        nd = len(shape)
        return pl.BlockSpec(shape, lambda b, _nd=nd: (0,) * _nd)

    SPB = 2
    out = pl.pallas_call(
        functools.partial(_stm_body, N=N, DIM=DIM, L=L, K=K, H=H, SPB=SPB),
        grid=(B // SPB,),
        in_specs=[
            pl.BlockSpec((SPB, N, DIM), lambda b: (b, 0, 0)),
            pl.BlockSpec((SPB, 1, N), lambda b: (b, 0, 0)),
            pl.BlockSpec((SPB, N, 1), lambda b: (b, 0, 0)),
            fixed((1, DIM)),
            fixed((1, 1)),
            fixed((DIM, DIM)),
            fixed((DIM, DIM)),
            fixed((DIM, DIM)),
            fixed((DIM, DIM)),
            fixed((1, DIM)),
            fixed((1, DIM)),
            fixed((1, DIM)),
            fixed((DIM, HID)),
            fixed((1, HID)),
            fixed((HID, DIM)),
            fixed((1, DIM)),
        ],
        out_specs=pl.BlockSpec((SPB, L, DIM), lambda b: (b, 0, 0)),
        out_shape=jax.ShapeDtypeStruct((B, L, DIM), jnp.float32),
        compiler_params=pltpu.CompilerParams(
            dimension_semantics=("parallel",)),
    )(xn, u3, sq, sw_row, sb2, wq, wk, wv, wo, bo2,
      n2g2, n2b2, w1, b1_2, w2, b2_2)
    return out


# speedup vs baseline: 10.9929x; 1.0006x over previous
import functools

import jax
import jax.numpy as jnp
from jax import lax
from jax.experimental import pallas as pl
from jax.experimental.pallas import tpu as pltpu

_L = 144
_K = 5
_H = 6


def _stm_body(xn_ref, u_ref, sq_ref, sw_ref, sb_ref,
              wq_ref, wk_ref, wv_ref, wo_ref, bo_ref,
              n2g_ref, n2b_ref, w1_ref, b1_ref, w2_ref, b2_ref,
              out_ref, *, N, DIM, L, K, H, SPB):
    HD = DIM // H
    SCALE = HD ** -0.5
    f32 = jnp.float32

    ii = lax.broadcasted_iota(jnp.int32, (N, N), 0)
    jj = lax.broadcasted_iota(jnp.int32, (N, N), 1)
    diag = ii == jj

    def _to_row(v_col):
        e = jnp.where(diag, jnp.broadcast_to(v_col, (N, N)),
                      jnp.float32(-jnp.inf))
        return jnp.max(e, axis=0, keepdims=True)

    for smp in range(SPB):
        _one_sample(smp, xn_ref, u_ref, sq_ref, sw_ref, sb_ref,
                    wq_ref, wk_ref, wv_ref, wo_ref, bo_ref,
                    n2g_ref, n2b_ref, w1_ref, b1_ref, w2_ref, b2_ref,
                    out_ref, N, DIM, L, K, H, HD, SCALE, f32,
                    ii, jj, _to_row)


def _one_sample(smp, xn_ref, u_ref, sq_ref, sw_ref, sb_ref,
                wq_ref, wk_ref, wv_ref, wo_ref, bo_ref,
                n2g_ref, n2b_ref, w1_ref, b1_ref, w2_ref, b2_ref,
                out_ref, N, DIM, L, K, H, HD, SCALE, f32,
                ii, jj, _to_row):
    diag = ii == jj
    xn = xn_ref[smp]
    sq_col = sq_ref[smp]
    sq_row = _to_row(sq_col)
    u_row = u_ref[smp]

    def _to_col(v_row):
        e = jnp.where(diag, jnp.broadcast_to(v_row, (N, N)),
                      jnp.float32(jnp.inf))
        return jnp.min(e, axis=-1, keepdims=True)

    ts_col = jnp.sum(xn * sw_ref[...], axis=-1, keepdims=True) + sb_ref[0, 0]

    g = lax.dot_general(xn, xn, (((1,), (1,)), ((), ())),
                        preferred_element_type=f32)
    d2 = sq_col + sq_row - 2.0 * g

    mns = [jnp.min(d2, axis=0, keepdims=True)]
    d2max = jnp.max(d2)
    for _ in range(K - 1):
        prev = mns[-1]
        mns.append(jnp.min(jnp.where(d2 > prev, d2, jnp.float32(jnp.inf)),
                           axis=0, keepdims=True))
    enc12 = jnp.sum((d2 == mns[0]).astype(f32)
                    + 1024.0 * (d2 == mns[1]).astype(f32),
                    axis=0, keepdims=True)
    enc34 = jnp.sum((d2 == mns[2]).astype(f32)
                    + 1024.0 * (d2 == mns[3]).astype(f32),
                    axis=0, keepdims=True)
    c2 = jnp.floor(enc12 * (1.0 / 1024.0))
    c1 = enc12 - 1024.0 * c2
    c4 = jnp.floor(enc34 * (1.0 / 1024.0))
    c3 = enc34 - 1024.0 * c4
    cum1 = c1
    cum2 = cum1 + c2
    cum3 = cum2 + c3
    cum4 = cum3 + c4

    def _pos(p):
        pf = jnp.float32(p)
        return jnp.where(pf < cum1, mns[0],
               jnp.where(pf < cum2, mns[1],
               jnp.where(pf < cum3, mns[2],
               jnp.where(pf < cum4, mns[3], mns[4]))))

    ms = []
    for p in range(K):
        s = jnp.sqrt(jnp.maximum(_pos(p), 0.0)) / (DIM ** 0.5)
        ms.append(s * s)
    acc = ((ms[0] + ms[4]) + ms[2]) + (ms[1] + ms[3])
    density_row = jnp.exp(-(acc / K)) + u_row * 1e-6
    density_col = _to_col(density_row)

    tmp = jnp.where(density_col > density_row, d2, d2max)
    dist_d2 = jnp.min(tmp, axis=0, keepdims=True)
    dist_row = jnp.sqrt(jnp.maximum(dist_d2, 0.0)) / (DIM ** 0.5)
    score_row = dist_row * density_row
    score_col = _to_col(score_row)

    gt = (score_col > score_row).astype(f32)
    tie = ((score_col == score_row) & (ii < jj)).astype(f32)
    rank_row = jnp.sum(gt + tie, axis=0, keepdims=True)
    sel_row = rank_row < L
    rank_col = _to_col(rank_row)
    sel_col = rank_col < L

    cand = jnp.where(sel_col,
                     jnp.sqrt(jnp.maximum(d2, 0.0)) / (DIM ** 0.5),
                     jnp.float32(jnp.inf))
    mind = jnp.min(cand, axis=0, keepdims=True)
    cid_row = jnp.min(jnp.where(cand == mind, rank_col, jnp.float32(N)),
                      axis=0, keepdims=True)
    cluster_row = jnp.where(sel_row, rank_row, cid_row)

    bias_row = _to_row(ts_col)
    w_row = jnp.exp(bias_row)
    iota_c = lax.broadcasted_iota(jnp.int32, (L, N), 0).astype(f32)
    a = (cluster_row == iota_c).astype(f32)
    allw = jnp.sum(a * w_row, axis=-1, keepdims=True) + 1e-6
    xw = xn * jnp.exp(ts_col)
    dnm = (((1,), (0,)), ((), ()))
    merged = lax.dot_general(a, xw, dnm, preferred_element_type=f32) * (1.0 / allw)

    dn = (((1,), (0,)), ((), ()))
    q = lax.dot_general(merged, wq_ref[...], dn, preferred_element_type=f32)
    k = lax.dot_general(xn, wk_ref[...], dn, preferred_element_type=f32)
    v = lax.dot_general(xn, wv_ref[...], dn, preferred_element_type=f32)
    outs = []
    for h in range(H):
        s = h * HD
        qh = q[:, s:s + HD]
        kh = k[:, s:s + HD]
        vh = v[:, s:s + HD]
        dots = lax.dot_general(qh, kh, (((1,), (1,)), ((), ())),
                               preferred_element_type=f32) * SCALE + bias_row
        mx = jnp.max(dots, axis=-1, keepdims=True)
        p = jnp.exp(dots - mx)
        attn = p * (1.0 / jnp.sum(p, axis=-1, keepdims=True))
        outs.append(lax.dot_general(attn, vh, dn, preferred_element_type=f32))
    att = jnp.concatenate(outs, axis=-1)
    att = lax.dot_general(att, wo_ref[...], dn,
                          preferred_element_type=f32) + bo_ref[...]

    feature = merged + att

    m2 = jnp.mean(feature, axis=-1, keepdims=True)
    v2 = jnp.mean((feature - m2) ** 2, axis=-1, keepdims=True)
    fn = (feature - m2) * (1.0 / jnp.sqrt(v2 + 1e-5)) * n2g_ref[...] + n2b_ref[...]
    hh = lax.dot_general(fn, w1_ref[...], dn,
                         preferred_element_type=f32) + b1_ref[...]
    ge = 0.5 * hh * (1.0 + lax.erf(hh * (2.0 ** -0.5)))
    y = feature + lax.dot_general(ge, w2_ref[...], dn,
                                  preferred_element_type=f32) + b2_ref[...]

    out_ref[smp] = y


def kernel(x, norm1_g, norm1_b, score_w, score_b, wq, wk, wv, wo, bo,
           norm2_g, norm2_b, w1, b1, w2, b2):
    B, N, DIM = x.shape
    L, K, H = _L, _K, _H
    HID = w1.shape[1]

    m = jnp.mean(x, axis=-1, keepdims=True)
    var = jnp.mean((x - m) ** 2, axis=-1, keepdims=True)
    xn = (x - m) / jnp.sqrt(var + 1e-5) * norm1_g + norm1_b
    sq = jnp.sum(xn * xn, axis=-1, keepdims=True)

    u = jax.random.uniform(jax.random.key(1), (B, N), jnp.float32)
    u3 = u.reshape(B, 1, N)

    sw_row = score_w.reshape(1, DIM)
    sb2 = score_b.reshape(1, 1)
    bo2 = bo.reshape(1, DIM)
    n2g2 = norm2_g.reshape(1, DIM)
    n2b2 = norm2_b.reshape(1, DIM)
    b1_2 = b1.reshape(1, HID)
    b2_2 = b2.reshape(1, DIM)

    def fixed(shape):
        nd = len(shape)
        return pl.BlockSpec(shape, lambda b, _nd=nd: (0,) * _nd)

    SPB = 2
    out = pl.pallas_call(
        functools.partial(_stm_body, N=N, DIM=DIM, L=L, K=K, H=H, SPB=SPB),
        grid=(B // SPB,),
        in_specs=[
            pl.BlockSpec((SPB, N, DIM), lambda b: (b, 0, 0)),
            pl.BlockSpec((SPB, 1, N), lambda b: (b, 0, 0)),
            pl.BlockSpec((SPB, N, 1), lambda b: (b, 0, 0)),
            fixed((1, DIM)),
            fixed((1, 1)),
            fixed((DIM, DIM)),
            fixed((DIM, DIM)),
            fixed((DIM, DIM)),
            fixed((DIM, DIM)),
            fixed((1, DIM)),
            fixed((1, DIM)),
            fixed((1, DIM)),
            fixed((DIM, HID)),
            fixed((1, HID)),
            fixed((HID, DIM)),
            fixed((1, DIM)),
        ],
        out_specs=pl.BlockSpec((SPB, L, DIM), lambda b: (b, 0, 0)),
        out_shape=jax.ShapeDtypeStruct((B, L, DIM), jnp.float32),
        compiler_params=pltpu.CompilerParams(
            dimension_semantics=("parallel",)),
    )(xn, u3, sq, sw_row, sb2, wq, wk, wv, wo, bo2,
      n2g2, n2b2, w1, b1_2, w2, b2_2)
    return out
